# baseline (speedup 1.0000x reference)
import functools

import jax
import jax.numpy as jnp
from jax import lax
from jax.experimental import pallas as pl
from jax.experimental.pallas import tpu as pltpu
from jax.experimental.pallas import tpu_sc as plsc

NC = 2
NS = 16
NW = NC * NS

CH = 128
K = 8
ROWS = CH * K


def _gather_rows(table, idx2d, n_rows, d):
    mesh = plsc.VectorSubcoreMesh(
        core_axis_name="c", subcore_axis_name="s", num_cores=NC, num_subcores=NS
    )
    rows_per_w = n_rows // NW
    iters = rows_per_w // ROWS

    @functools.partial(
        pl.kernel,
        out_type=jax.ShapeDtypeStruct((n_rows // CH, CH, d), jnp.float32),
        mesh=mesh,
        scratch_types=[
            pltpu.VMEM((K, CH), jnp.int32),
            pltpu.VMEM((K, CH, d), jnp.float32),
            pltpu.SemaphoreType.DMA,
        ],
        compiler_params=pltpu.CompilerParams(use_tc_tiling_on_sc=False),
    )
    def body(table_hbm, idx_hbm, out_hbm, idx_v, rows_v, sem):
        wid = lax.axis_index("s") * NC + lax.axis_index("c")
        chunk0 = wid * (rows_per_w // CH)

        def step(i, carry):
            c0 = chunk0 + i * K
            pltpu.sync_copy(idx_hbm.at[pl.ds(c0, K)], idx_v)
            copies = []
            for j in range(K):
                copies.append(
                    pltpu.async_copy(
                        table_hbm.at[idx_v.at[j]],
                        rows_v.at[j],
                        sem,
                    )
                )
            for c in copies:
                c.wait()
            pltpu.sync_copy(rows_v, out_hbm.at[pl.ds(c0, K)])
            return carry

        lax.fori_loop(0, iters, step, 0)

    return body(table, idx2d)


def kernel(indices, table):
    b, p = indices.shape
    v, d = table.shape
    n = b * p
    idx2d = indices.reshape(n // CH, CH).astype(jnp.int32)
    out = _gather_rows(table, idx2d, n, d)
    return out.reshape(b, p, d)


# speedup vs baseline: 1.2242x; 1.2242x over previous
import functools

import jax
import jax.numpy as jnp
from jax import lax
from jax.experimental import pallas as pl
from jax.experimental.pallas import tpu as pltpu
from jax.experimental.pallas import tpu_sc as plsc

NC = 2
NS = 16
NW = NC * NS

CH = 128
K = 4
DP = 56

VB = 2048


def _widen(table_t, vp):
    d, v = table_t.shape
    nfull = v // VB
    rem = vp - nfull * VB

    def body(in_hbm, out_hbm, in_v, out_v, sem_in, sem_out):
        i = pl.program_id(0)
        v0 = i * VB

        def work(w):
            pltpu.async_copy(
                in_hbm.at[:, pl.ds(v0, w)], in_v.at[:, pl.ds(0, w)], sem_in
            ).wait()
            t = in_v[...]
            tt = jnp.transpose(t, (1, 0))
            pad = jnp.zeros((VB, 128 - d), dtype=tt.dtype)
            out_v[...] = jnp.concatenate([tt, pad], axis=1)
            pltpu.async_copy(
                out_v.at[pl.ds(0, w)], out_hbm.at[pl.ds(v0, w)], sem_out
            ).wait()

        @pl.when(i < nfull)
        def _full():
            work(VB)

        @pl.when(i == nfull)
        def _rem():
            work(rem)

    return pl.pallas_call(
        body,
        grid=(nfull + (1 if rem else 0),),
        in_specs=[pl.BlockSpec(memory_space=pltpu.MemorySpace.HBM)],
        out_specs=pl.BlockSpec(memory_space=pltpu.MemorySpace.HBM),
        scratch_shapes=[
            pltpu.VMEM((d, VB), jnp.float32),
            pltpu.VMEM((VB, 128), jnp.float32),
            pltpu.SemaphoreType.DMA,
            pltpu.SemaphoreType.DMA,
        ],
        out_shape=jax.ShapeDtypeStruct((vp, 128), jnp.float32),
    )(table_t)


def _gather_rows(table128, idx2d, n_rows, d):
    mesh = plsc.VectorSubcoreMesh(
        core_axis_name="c", subcore_axis_name="s", num_cores=NC, num_subcores=NS
    )
    rows_per_w = n_rows // NW
    chunks_per_w = rows_per_w // CH
    iters = chunks_per_w // K

    @functools.partial(
        pl.kernel,
        out_type=jax.ShapeDtypeStruct((n_rows // CH, CH, DP), jnp.float32),
        mesh=mesh,
        scratch_types=[
            pltpu.VMEM((K, CH), jnp.int32),
            pltpu.VMEM((K, CH, 128), jnp.float32),
            pltpu.SemaphoreType.DMA,
        ],
        compiler_params=pltpu.CompilerParams(use_tc_tiling_on_sc=False),
    )
    def body(table_hbm, idx_hbm, out_hbm, idx_v, rows_v, sem):
        wid = lax.axis_index("s") * NC + lax.axis_index("c")
        chunk0 = wid * chunks_per_w

        def step(i, carry):
            c0 = chunk0 + i * K
            pltpu.sync_copy(idx_hbm.at[pl.ds(c0, K)], idx_v)
            copies = []
            for j in range(K):
                copies.append(
                    pltpu.async_copy(
                        table_hbm.at[idx_v.at[j]],
                        rows_v.at[j],
                        sem,
                    )
                )
            for c in copies:
                c.wait()
            pltpu.sync_copy(
                rows_v.at[:, :, pl.ds(0, DP)], out_hbm.at[pl.ds(c0, K)]
            )
            return carry

        lax.fori_loop(0, iters, step, 0)

    return body(table128, idx2d)


def kernel(indices, table):
    b, p = indices.shape
    v, d = table.shape
    n = b * p
    vp = (v + 127) // 128 * 128
    table128 = _widen(table.T, vp)
    idx2d = indices.reshape(n // CH, CH).astype(jnp.int32)
    out = _gather_rows(table128, idx2d, n, d)
    return out[:, :, :d].reshape(b, p, d)


# speedup vs baseline: 2.3491x; 1.9190x over previous
import functools

import jax
import jax.numpy as jnp
from jax import lax
from jax.experimental import pallas as pl
from jax.experimental.pallas import tpu as pltpu
from jax.experimental.pallas import tpu_sc as plsc

NC = 2
NS = 16
NW = NC * NS

CH = 128
K = 4
DP = 56

VB = 4096


def _widen(table_t, vp):
    d, v = table_t.shape

    def body(in_ref, out_ref):
        tt = jnp.transpose(in_ref[...], (1, 0))
        pad = jnp.zeros((VB, 128 - d), dtype=tt.dtype)
        out_ref[...] = jnp.concatenate([tt, pad], axis=1)

    return pl.pallas_call(
        body,
        grid=(pl.cdiv(vp, VB),),
        in_specs=[pl.BlockSpec((d, VB), lambda i: (0, i))],
        out_specs=pl.BlockSpec((VB, 128), lambda i: (i, 0)),
        out_shape=jax.ShapeDtypeStruct((vp, 128), jnp.float32),
    )(table_t)


def _gather_rows(table128, idx2d, n_rows, d):
    mesh = plsc.VectorSubcoreMesh(
        core_axis_name="c", subcore_axis_name="s", num_cores=NC, num_subcores=NS
    )
    rows_per_w = n_rows // NW
    chunks_per_w = rows_per_w // CH
    iters = chunks_per_w // K

    @functools.partial(
        pl.kernel,
        out_type=jax.ShapeDtypeStruct((n_rows // CH, CH, DP), jnp.float32),
        mesh=mesh,
        scratch_types=[
            pltpu.VMEM((K, CH), jnp.int32),
            pltpu.VMEM((K, CH, 128), jnp.float32),
            pltpu.SemaphoreType.DMA,
        ],
        compiler_params=pltpu.CompilerParams(use_tc_tiling_on_sc=False),
    )
    def body(table_hbm, idx_hbm, out_hbm, idx_v, rows_v, sem):
        wid = lax.axis_index("s") * NC + lax.axis_index("c")
        chunk0 = wid * chunks_per_w

        def step(i, carry):
            c0 = chunk0 + i * K
            pltpu.sync_copy(idx_hbm.at[pl.ds(c0, K)], idx_v)
            copies = []
            for j in range(K):
                copies.append(
                    pltpu.async_copy(
                        table_hbm.at[idx_v.at[j]],
                        rows_v.at[j],
                        sem,
                    )
                )
            for c in copies:
                c.wait()
            pltpu.sync_copy(
                rows_v.at[:, :, pl.ds(0, DP)], out_hbm.at[pl.ds(c0, K)]
            )
            return carry

        lax.fori_loop(0, iters, step, 0)

    return body(table128, idx2d)


def kernel(indices, table):
    b, p = indices.shape
    v, d = table.shape
    n = b * p
    vp = (v + 127) // 128 * 128
    table128 = _widen(table.T, vp)
    idx2d = indices.reshape(n // CH, CH).astype(jnp.int32)
    out = _gather_rows(table128, idx2d, n, d)
    return out[:, :, :d].reshape(b, p, d)


# speedup vs baseline: 2.6227x; 1.1165x over previous
import functools

import jax
import jax.numpy as jnp
from jax import lax
from jax.experimental import pallas as pl
from jax.experimental.pallas import tpu as pltpu
from jax.experimental.pallas import tpu_sc as plsc

NC = 2
NS = 16
NW = NC * NS

CH = 128
K = 4
DP = 56

VB = 16384


def _widen(table_t, vp):
    d, v = table_t.shape

    def body(in_ref, out_ref):
        tt = jnp.transpose(in_ref[...], (1, 0))
        pad = jnp.zeros((VB, 128 - d), dtype=tt.dtype)
        out_ref[...] = jnp.concatenate([tt, pad], axis=1)

    return pl.pallas_call(
        body,
        grid=(pl.cdiv(vp, VB),),
        in_specs=[pl.BlockSpec((d, VB), lambda i: (0, i))],
        out_specs=pl.BlockSpec((VB, 128), lambda i: (i, 0)),
        out_shape=jax.ShapeDtypeStruct((vp, 128), jnp.float32),
    )(table_t)


def _gather_rows(table128, idx2d, n_rows, d):
    mesh = plsc.VectorSubcoreMesh(
        core_axis_name="c", subcore_axis_name="s", num_cores=NC, num_subcores=NS
    )
    rows_per_w = n_rows // NW
    chunks_per_w = rows_per_w // CH
    iters = chunks_per_w // K

    @functools.partial(
        pl.kernel,
        out_type=jax.ShapeDtypeStruct((n_rows // CH, CH, DP), jnp.float32),
        mesh=mesh,
        scratch_types=[
            pltpu.VMEM((K, CH), jnp.int32),
            pltpu.VMEM((K, CH, 128), jnp.float32),
            pltpu.SemaphoreType.DMA,
        ],
        compiler_params=pltpu.CompilerParams(use_tc_tiling_on_sc=False),
    )
    def body(table_hbm, idx_hbm, out_hbm, idx_v, rows_v, sem):
        wid = lax.axis_index("s") * NC + lax.axis_index("c")
        chunk0 = wid * chunks_per_w

        def step(i, carry):
            c0 = chunk0 + i * K
            pltpu.sync_copy(idx_hbm.at[pl.ds(c0, K)], idx_v)
            copies = []
            for j in range(K):
                copies.append(
                    pltpu.async_copy(
                        table_hbm.at[idx_v.at[j]],
                        rows_v.at[j],
                        sem,
                    )
                )
            for c in copies:
                c.wait()
            pltpu.sync_copy(
                rows_v.at[:, :, pl.ds(0, DP)], out_hbm.at[pl.ds(c0, K)]
            )
            return carry

        lax.fori_loop(0, iters, step, 0)

    return body(table128, idx2d)


def kernel(indices, table):
    b, p = indices.shape
    v, d = table.shape
    n = b * p
    vp = (v + 127) // 128 * 128
    table128 = _widen(table.T, vp)
    idx2d = indices.reshape(n // CH, CH).astype(jnp.int32)
    out = _gather_rows(table128, idx2d, n, d)
    return out[:, :, :d].reshape(b, p, d)


# speedup vs baseline: 2.6405x; 1.0068x over previous
import functools

import jax
import jax.numpy as jnp
from jax import lax
from jax.experimental import pallas as pl
from jax.experimental.pallas import tpu as pltpu
from jax.experimental.pallas import tpu_sc as plsc

NC = 2
NS = 16
NW = NC * NS

CH = 128
K = 4
DP = 56

VB = 32768


def _widen(table_t, vp):
    d, v = table_t.shape

    def body(in_ref, out_ref):
        tt = jnp.transpose(in_ref[...], (1, 0))
        pad = jnp.zeros((VB, 128 - d), dtype=tt.dtype)
        out_ref[...] = jnp.concatenate([tt, pad], axis=1)

    return pl.pallas_call(
        body,
        grid=(pl.cdiv(vp, VB),),
        in_specs=[pl.BlockSpec((d, VB), lambda i: (0, i))],
        out_specs=pl.BlockSpec((VB, 128), lambda i: (i, 0)),
        out_shape=jax.ShapeDtypeStruct((vp, 128), jnp.float32),
    )(table_t)


def _gather_rows(table128, idx2d, n_rows, d):
    mesh = plsc.VectorSubcoreMesh(
        core_axis_name="c", subcore_axis_name="s", num_cores=NC, num_subcores=NS
    )
    rows_per_w = n_rows // NW
    chunks_per_w = rows_per_w // CH
    iters = chunks_per_w // K

    @functools.partial(
        pl.kernel,
        out_type=jax.ShapeDtypeStruct((n_rows // CH, CH, DP), jnp.float32),
        mesh=mesh,
        scratch_types=[
            pltpu.VMEM((K, CH), jnp.int32),
            pltpu.VMEM((K, CH, 128), jnp.float32),
            pltpu.SemaphoreType.DMA,
        ],
        compiler_params=pltpu.CompilerParams(use_tc_tiling_on_sc=False),
    )
    def body(table_hbm, idx_hbm, out_hbm, idx_v, rows_v, sem):
        wid = lax.axis_index("s") * NC + lax.axis_index("c")
        chunk0 = wid * chunks_per_w

        def step(i, carry):
            c0 = chunk0 + i * K
            pltpu.sync_copy(idx_hbm.at[pl.ds(c0, K)], idx_v)
            copies = []
            for j in range(K):
                copies.append(
                    pltpu.async_copy(
                        table_hbm.at[idx_v.at[j]],
                        rows_v.at[j],
                        sem,
                    )
                )
            for c in copies:
                c.wait()
            pltpu.sync_copy(
                rows_v.at[:, :, pl.ds(0, DP)], out_hbm.at[pl.ds(c0, K)]
            )
            return carry

        lax.fori_loop(0, iters, step, 0)

    return body(table128, idx2d)


def kernel(indices, table):
    b, p = indices.shape
    v, d = table.shape
    n = b * p
    vp = (v + 127) // 128 * 128
    table128 = _widen(table.T, vp)
    idx2d = indices.reshape(n // CH, CH).astype(jnp.int32)
    out = _gather_rows(table128, idx2d, n, d)
    return out[:, :, :d].reshape(b, p, d)


# speedup vs baseline: 3.0471x; 1.1540x over previous
import functools

import jax
import jax.numpy as jnp
from jax import lax
from jax.experimental import pallas as pl
from jax.experimental.pallas import tpu as pltpu
from jax.experimental.pallas import tpu_sc as plsc

NC = 2
NS = 16
NW = NC * NS

CH = 128
K = 4

VB = 32768


def _widen(table_t, vp):
    d, v = table_t.shape

    def body(in_ref, out_ref):
        tt = jnp.transpose(in_ref[...], (1, 0))
        pad = jnp.zeros((VB, 128 - d), dtype=tt.dtype)
        out_ref[...] = jnp.concatenate([tt, pad], axis=1)

    return pl.pallas_call(
        body,
        grid=(pl.cdiv(vp, VB),),
        in_specs=[pl.BlockSpec((d, VB), lambda i: (0, i))],
        out_specs=pl.BlockSpec((VB, 128), lambda i: (i, 0)),
        out_shape=jax.ShapeDtypeStruct((vp, 128), jnp.float32),
    )(table_t)


def _gather_t(table128, idxp, b, p, d):
    n = b * p
    nchunks = n // CH
    chunks_per_p = b // CH
    mesh = plsc.VectorSubcoreMesh(
        core_axis_name="c", subcore_axis_name="s", num_cores=NC, num_subcores=NS
    )
    chunks_per_w = nchunks // NW
    iters = chunks_per_w // K

    @functools.partial(
        pl.kernel,
        out_type=jax.ShapeDtypeStruct((p, d, b), jnp.float32),
        mesh=mesh,
        scratch_types=[
            pltpu.VMEM((K, CH), jnp.int32),
            pltpu.VMEM((K, CH, 128), jnp.float32),
            pltpu.VMEM((d, CH), jnp.float32),
            pltpu.SemaphoreType.DMA,
        ],
        compiler_params=pltpu.CompilerParams(
            use_tc_tiling_on_sc=True, needs_layout_passes=False
        ),
    )
    def body(table_hbm, idx_hbm, out_hbm, idx_v, rows_v, t_v, sem):
        wid = lax.axis_index("s") * NC + lax.axis_index("c")
        chunk0 = wid * chunks_per_w
        lane = lax.iota(jnp.int32, 16)

        def step(i, carry):
            c0 = chunk0 + i * K
            pltpu.sync_copy(idx_hbm.at[pl.ds(c0, K)], idx_v)
            copies = []
            for j in range(K):
                copies.append(
                    pltpu.async_copy(
                        table_hbm.at[idx_v.at[j]],
                        rows_v.at[j],
                        sem,
                    )
                )
            for j in range(K):
                copies[j].wait()
                c = c0 + j
                pj = c // chunks_per_p
                nb0 = (c % chunks_per_p) * CH

                def word(w, carry2):
                    wv = jnp.full((16,), 0, jnp.int32) + w
                    for r0 in range(0, CH, 16):
                        vals = plsc.load_gather(
                            rows_v.at[j], [lane + r0, wv]
                        )
                        t_v[w, pl.ds(r0, 16)] = vals
                    return carry2

                lax.fori_loop(0, d, word, 0)
                pltpu.sync_copy(
                    t_v,
                    out_hbm.at[pj, :, pl.ds(pl.multiple_of(nb0, CH), CH)],
                )
            return carry

        lax.fori_loop(0, iters, step, 0)

    return body(table128, idxp)


def kernel(indices, table):
    b, p = indices.shape
    v, d = table.shape
    vp = (v + 127) // 128 * 128
    table128 = _widen(table.T, vp)
    idxp = indices.T.reshape(b * p // CH, CH).astype(jnp.int32)
    out_t = _gather_t(table128, idxp, b, p, d)
    return jnp.transpose(out_t, (2, 0, 1))
